# baseline (speedup 1.0000x reference)
import jax
import jax.numpy as jnp
from jax.experimental import pallas as pl
from jax.experimental.pallas import tpu as pltpu

N = 1024
F_IN = 128
HID = 512
H = 8
DH = HID // H
L = 6
NODES_PER_SUB = 8
S = N // NODES_PER_SUB

CH = 256
NC = N // CH
SBC = CH // NODES_PER_SUB


def _gat_body(x_ref, adj_ref, lw_ref, lb_ref, w_ref, as_ref, ad_ref,
              node_ref, sub_ref, prob_ref, abuf, obuf, in_sem, out_sem):
    f32 = jnp.float32

    def in_copy(c, slot):
        return pltpu.make_async_copy(adj_ref.at[pl.ds(c * CH, CH), :],
                                     abuf.at[slot], in_sem.at[slot])

    def out_copy(c, slot):
        return pltpu.make_async_copy(obuf.at[slot],
                                     node_ref.at[pl.ds(c * CH, CH), :],
                                     out_sem.at[slot])

    in_copy(0, 0).start()
    in_copy(1, 1).start()

    h0 = jnp.dot(x_ref[...], lw_ref[...], preferred_element_type=f32) + lb_ref[...]
    h = jnp.dot(h0, w_ref[0], preferred_element_type=f32)
    a_st = as_ref[0].T
    a_d = ad_ref[0]
    log2e = 1.4426950408889634
    ones = jnp.ones((N, 1), f32)
    hs1 = []
    s_all = []
    d_all = []
    for hd in range(H):
        hsl = h[:, hd * DH:(hd + 1) * DH]
        s_all.append(jnp.dot(hsl, a_st[:, hd:hd + 1], preferred_element_type=f32))
        d_all.append(jax.lax.dot_general(a_d[hd:hd + 1, :], hsl,
                                         (((1,), (1,)), ((), ())),
                                         preferred_element_type=f32))
        hs1.append(jnp.concatenate([hsl, ones], axis=1))

    r = jax.lax.broadcasted_iota(jnp.int32, (SBC, CH), 0)
    c2 = jax.lax.broadcasted_iota(jnp.int32, (SBC, CH), 1)
    pool = jnp.where(c2 // NODES_PER_SUB == r, 1.0 / NODES_PER_SUB, 0.0).astype(f32)

    for c in range(NC):
        slot = c % 2
        in_copy(c, slot).wait()
        if c >= 2:
            out_copy(c - 2, slot).wait()
        adjc = abuf[slot]
        ob = obuf.at[slot]
        for hd in range(H):
            s2 = s_all[hd][c * CH:(c + 1) * CH, :] * log2e
            d2 = d_all[hd] * log2e
            p = jnp.exp2(jnp.maximum(s2 + d2, 0.2 * s2 + 0.2 * d2)) * adjc
            u = jnp.dot(p, hs1[hd], preferred_element_type=f32)
            o = u[:, :DH] / u[:, DH:]
            ob[:, hd * DH:(hd + 1) * DH] = jnp.where(o > 0, o, jnp.exp(o) - 1.0)
        sub_ref[c * SBC:(c + 1) * SBC, :] = jnp.dot(pool, obuf[slot],
                                                    preferred_element_type=f32)
        out_copy(c, slot).start()
        if c + 2 < NC:
            in_copy(c + 2, slot).start()

    out_copy(NC - 2, (NC - 2) % 2).wait()
    out_copy(NC - 1, (NC - 1) % 2).wait()
    prob_ref[...] = jnp.ones((S, 1), f32)


def kernel(x, adj, lin_w, lin_b, gat_w, gat_a_src, gat_a_dst, cls_w, cls_b):
    f32 = jnp.float32
    node, sub, prob = pl.pallas_call(
        _gat_body,
        grid=(1,),
        in_specs=[
            pl.BlockSpec((N, F_IN), lambda i: (0, 0)),
            pl.BlockSpec(memory_space=pl.ANY),
            pl.BlockSpec((F_IN, HID), lambda i: (0, 0)),
            pl.BlockSpec((1, HID), lambda i: (0, 0)),
            pl.BlockSpec((1, HID, HID), lambda i: (L - 1, 0, 0)),
            pl.BlockSpec((1, H, DH), lambda i: (L - 1, 0, 0)),
            pl.BlockSpec((1, H, DH), lambda i: (L - 1, 0, 0)),
        ],
        out_specs=(
            pl.BlockSpec(memory_space=pl.ANY),
            pl.BlockSpec((S, HID), lambda i: (0, 0)),
            pl.BlockSpec((S, 1), lambda i: (0, 0)),
        ),
        out_shape=(
            jax.ShapeDtypeStruct((N, HID), f32),
            jax.ShapeDtypeStruct((S, HID), f32),
            jax.ShapeDtypeStruct((S, 1), f32),
        ),
        scratch_shapes=[
            pltpu.VMEM((2, CH, N), f32),
            pltpu.VMEM((2, CH, HID), f32),
            pltpu.SemaphoreType.DMA((2,)),
            pltpu.SemaphoreType.DMA((2,)),
        ],
    )(x, adj, lin_w, lin_b.reshape(1, HID), gat_w, gat_a_src, gat_a_dst)
    return (prob, node, sub)
